# baseline (speedup 1.0000x reference)
import jax
import jax.numpy as jnp
from jax.experimental import pallas as pl
from jax.experimental.pallas import tpu as pltpu

EPS = 1e-5
K = 16
P = 1024
BIGF = 1e30
_SQRT1_2 = 0.7071067811865476


def _gelu(x):
    return 0.5 * x * (1.0 + jax.lax.erf(x * _SQRT1_2))


RG = 8
UN = 4
NCH = 8


def _frame_kernel(x_ref, xT_ref, waT_ref, wbT_ref, s1_ref, c1_ref, w2T_ref,
                  s2_ref, c2_ref, pw1aT_ref, pw1bT_ref, pb1_ref, pg1_ref,
                  pe1_ref, pw2T_ref, pb2_ref, pg2_ref, pe2_ref, o_ref,
                  d2_s, pidx_s, xj0_s, xj1_s, xj2_s, xj3_s):
    f32 = jnp.float32
    x = x_ref[0]
    xT = xT_ref[0]

    lane8 = jax.lax.broadcasted_iota(jnp.int32, (1, 8), 1)
    sub8 = jax.lax.broadcasted_iota(jnp.int32, (8, 1), 0)
    xyz = jnp.where(lane8 < 3, x, 0.0)
    xyzT = jnp.where(sub8 < 3, xT, 0.0)

    dot = jnp.dot(xyz, xyzT, preferred_element_type=f32)
    sqr = jnp.sum(xyz * xyz, axis=1, keepdims=True)
    sqc = jnp.sum(xyzT * xyzT, axis=0, keepdims=True)
    d2_s[...] = jnp.maximum(sqr + sqc - 2.0 * dot, 0.0)
    pidx_s[...] = jnp.full((P, 1), -1.0, f32)

    axi = jnp.dot(x, waT_ref[...], preferred_element_type=f32)
    s1 = s1_ref[...]
    c1 = c1_ref[...]
    s2 = s2_ref[...]
    c2 = c2_ref[...]
    wbT = wbT_ref[...]
    w2T = w2T_ref[...]

    lane_f = jax.lax.broadcasted_iota(jnp.int32, (1, 128), 1).astype(f32)
    gidx = [lane_f + (128.0 * c) for c in range(NCH)]
    xtc = [[xT_ref[0, ch:ch + 1, 128 * c:128 * (c + 1)] for c in range(NCH)]
           for ch in range(4)]

    def one_group(base):
        pidx = pidx_s[pl.ds(base, RG), :]
        m = jnp.full((RG, 128), BIGF, f32)
        ri = jnp.full((RG, 128), float(P), f32)
        rx0 = jnp.zeros((RG, 128), f32)
        rx1 = jnp.zeros((RG, 128), f32)
        rx2 = jnp.zeros((RG, 128), f32)
        rx3 = jnp.zeros((RG, 128), f32)
        for c in range(NCH):
            g = d2_s[pl.ds(base, RG), 128 * c:128 * (c + 1)]
            g = jnp.where(gidx[c] == pidx, BIGF, g)
            d2_s[pl.ds(base, RG), 128 * c:128 * (c + 1)] = g
            upd = g < m
            m = jnp.where(upd, g, m)
            ri = jnp.where(upd, gidx[c], ri)
            rx0 = jnp.where(upd, xtc[0][c], rx0)
            rx1 = jnp.where(upd, xtc[1][c], rx1)
            rx2 = jnp.where(upd, xtc[2][c], rx2)
            rx3 = jnp.where(upd, xtc[3][c], rx3)
        v = jnp.min(m, axis=1, keepdims=True)
        idx = jnp.min(jnp.where(m == v, ri, float(P)), axis=1, keepdims=True)
        selw = ri == idx
        pidx_s[pl.ds(base, RG), :] = idx
        xj0_s[pl.ds(base, RG), :] = jnp.sum(jnp.where(selw, rx0, 0.0), axis=1, keepdims=True)
        xj1_s[pl.ds(base, RG), :] = jnp.sum(jnp.where(selw, rx1, 0.0), axis=1, keepdims=True)
        xj2_s[pl.ds(base, RG), :] = jnp.sum(jnp.where(selw, rx2, 0.0), axis=1, keepdims=True)
        xj3_s[pl.ds(base, RG), :] = jnp.sum(jnp.where(selw, rx3, 0.0), axis=1, keepdims=True)

    def rg_body(r, acc):
        base0 = pl.multiple_of(r * (RG * UN), RG * UN)
        for u in range(UN):
            one_group(base0 + u * RG)
        return acc

    def body(_, carry):
        ymax, ymin = carry
        jax.lax.fori_loop(0, P // (RG * UN), rg_body, 0)
        h1p = (axi + xj0_s[...] * wbT[0:1] + xj1_s[...] * wbT[1:2]
               + xj2_s[...] * wbT[2:3] + xj3_s[...] * wbT[3:4])
        h1 = _gelu(h1p * s1 + c1)
        y2 = jnp.dot(h1, w2T, preferred_element_type=f32) * s2 + c2
        return jnp.maximum(ymax, y2), jnp.minimum(ymin, y2)

    ymax0 = jnp.full((P, 128), -BIGF, f32)
    ymin0 = jnp.full((P, 128), BIGF, f32)
    ymax, ymin = jax.lax.fori_loop(0, K, body, (ymax0, ymin0))
    local = jnp.maximum(_gelu(ymax), _gelu(ymin))

    y = (jnp.dot(local, pw1aT_ref[...], preferred_element_type=f32)
         + jnp.dot(x, pw1bT_ref[...], preferred_element_type=f32)
         + pb1_ref[...])
    m = y.mean(axis=1, keepdims=True)
    va = ((y - m) ** 2).mean(axis=1, keepdims=True)
    y = _gelu((y - m) / jnp.sqrt(va + EPS) * pg1_ref[...] + pe1_ref[...])
    z = jnp.dot(y, pw2T_ref[...], preferred_element_type=f32) + pb2_ref[...]
    m = z.mean(axis=1, keepdims=True)
    va = ((z - m) ** 2).mean(axis=1, keepdims=True)
    z = _gelu((z - m) / jnp.sqrt(va + EPS) * pg2_ref[...] + pe2_ref[...])

    o_ref[0, :, 0:256] = jnp.max(z, axis=0, keepdims=True)
    o_ref[0, :, 256:512] = jnp.sum(z, axis=0, keepdims=True) * (1.0 / P)


def _temporal_kernel(pf_ref, projT_ref, projb_ref, wc_ref, cb_ref, bs_ref,
                     bb_ref, hw1T_ref, hb1_ref, hw2T_ref, hb2_ref, o_ref):
    f32 = jnp.float32
    pf = pf_ref[...]
    hs = jnp.dot(pf, projT_ref[...], preferred_element_type=f32) + projb_ref[...]
    rmod = jax.lax.broadcasted_iota(jnp.int32, (64, 1), 0) % 16

    def conv_block(lay, xin):
        w0 = wc_ref[3 * lay]
        w1 = wc_ref[3 * lay + 1]
        w2 = wc_ref[3 * lay + 2]
        xm1 = jnp.where(rmod == 0, 0.0,
                        jnp.concatenate([jnp.zeros((1, 256), f32), xin[:-1]], axis=0))
        xp1 = jnp.where(rmod == 15, 0.0,
                        jnp.concatenate([xin[1:], jnp.zeros((1, 256), f32)], axis=0))
        y = (jnp.dot(xm1, w0, preferred_element_type=f32)
             + jnp.dot(xin, w1, preferred_element_type=f32)
             + jnp.dot(xp1, w2, preferred_element_type=f32)
             + cb_ref[lay:lay + 1])
        return _gelu(y * bs_ref[lay:lay + 1] + bb_ref[lay:lay + 1])

    hs = conv_block(0, hs)
    hs = hs + conv_block(1, hs)
    hs = hs + conv_block(2, hs)
    hs = hs + conv_block(3, hs)

    parts = [jnp.max(hs[16 * b:16 * (b + 1)], axis=0, keepdims=True)
             for b in range(4)]
    mx = jnp.concatenate(parts, axis=0)
    h = _gelu(jnp.dot(mx, hw1T_ref[...], preferred_element_type=f32) + hb1_ref[...])
    o_ref[...] = jnp.dot(h, hw2T_ref[...], preferred_element_type=f32) + hb2_ref[...]


def kernel(x_pt, ec_w1, ec_g1, ec_b1, ec_w2, ec_g2, ec_b2, pm_w1, pm_b1,
           pm_g1, pm_be1, pm_w2, pm_b2, pm_g2, pm_be2, proj_w, proj_b,
           c1_w, c1_b, bn1_g, bn1_b, c2_w, c2_b, bn2_g, bn2_b, c3_w, c3_b,
           bn3_g, bn3_b, c4_w, c4_b, bn4_g, bn4_b, h_w1, h_b1, h_w2, h_b2):
    B, T, Pn, C = x_pt.shape
    BT = B * T
    f32 = jnp.float32
    x = x_pt.reshape(BT, Pn, C)
    x8 = jnp.concatenate([x, jnp.zeros((BT, Pn, 8 - C), f32)], axis=-1)
    xT8 = jnp.transpose(x8, (0, 2, 1))

    rbn = 1.0 / jnp.sqrt(jnp.float32(1.0 + EPS))
    waT = (ec_w1[:, :4] - ec_w1[:, 4:]).T
    waT = jnp.concatenate([waT, jnp.zeros((4, 64), f32)], axis=0)
    wbT = jnp.concatenate([ec_w1[:, 4:].T, jnp.zeros((4, 64), f32)], axis=0)
    s1 = (ec_g1 * rbn)[None]
    c1 = ec_b1[None]
    w2T = ec_w2.T
    s2 = (ec_g2 * rbn)[None]
    c2 = ec_b2[None]
    pw1aT = pm_w1[:, :128].T
    pw1bT = jnp.concatenate([pm_w1[:, 128:].T, jnp.zeros((4, 128), f32)], axis=0)

    rep = lambda s: pl.BlockSpec(s, lambda f: tuple([0] * len(s)))
    per_frame = pl.pallas_call(
        _frame_kernel,
        grid=(BT,),
        in_specs=[
            pl.BlockSpec((1, Pn, 8), lambda f: (f, 0, 0)),
            pl.BlockSpec((1, 8, Pn), lambda f: (f, 0, 0)),
            rep((8, 64)), rep((8, 64)), rep((1, 64)), rep((1, 64)),
            rep((64, 128)), rep((1, 128)), rep((1, 128)),
            rep((128, 128)), rep((8, 128)), rep((1, 128)), rep((1, 128)),
            rep((1, 128)),
            rep((128, 256)), rep((1, 256)), rep((1, 256)), rep((1, 256)),
        ],
        out_specs=pl.BlockSpec((1, 1, 512), lambda f: (f, 0, 0)),
        out_shape=jax.ShapeDtypeStruct((BT, 1, 512), f32),
        scratch_shapes=[
            pltpu.VMEM((Pn, Pn), f32),
            pltpu.VMEM((Pn, 1), f32),
            pltpu.VMEM((Pn, 1), f32),
            pltpu.VMEM((Pn, 1), f32),
            pltpu.VMEM((Pn, 1), f32),
            pltpu.VMEM((Pn, 1), f32),
        ],
    )(x8, xT8, waT, wbT, s1, c1, w2T, s2, c2,
      pw1aT, pw1bT, pm_b1[None], pm_g1[None], pm_be1[None],
      pm_w2.T, pm_b2[None], pm_g2[None], pm_be2[None])
    per_frame = per_frame.reshape(BT, 512)

    wc = jnp.stack([w[:, :, dt].T for w in (c1_w, c2_w, c3_w, c4_w)
                    for dt in range(3)], axis=0)
    cb = jnp.stack([c1_b, c2_b, c3_b, c4_b], axis=0)
    bs = jnp.stack([bn1_g, bn2_g, bn3_g, bn4_g], axis=0) * rbn
    bb = jnp.stack([bn1_b, bn2_b, bn3_b, bn4_b], axis=0)

    out = pl.pallas_call(
        _temporal_kernel,
        in_specs=[
            pl.BlockSpec((BT, 512), lambda: (0, 0)),
            pl.BlockSpec((512, 256), lambda: (0, 0)),
            pl.BlockSpec((1, 256), lambda: (0, 0)),
            pl.BlockSpec((12, 256, 256), lambda: (0, 0, 0)),
            pl.BlockSpec((4, 256), lambda: (0, 0)),
            pl.BlockSpec((4, 256), lambda: (0, 0)),
            pl.BlockSpec((4, 256), lambda: (0, 0)),
            pl.BlockSpec((256, 128), lambda: (0, 0)),
            pl.BlockSpec((1, 128), lambda: (0, 0)),
            pl.BlockSpec((128, 32), lambda: (0, 0)),
            pl.BlockSpec((1, 32), lambda: (0, 0)),
        ],
        out_specs=pl.BlockSpec((4, 32), lambda: (0, 0)),
        out_shape=jax.ShapeDtypeStruct((4, 32), f32),
    )(per_frame, proj_w.T, proj_b[None], wc, cb, bs, bb,
      h_w1.T, h_b1[None],
      jnp.concatenate([h_w2.T, jnp.zeros((128, 7), f32)], axis=1),
      jnp.concatenate([h_b2, jnp.zeros((7,), f32)])[None])
    return out[:, :25]


# speedup vs baseline: 3.1603x; 3.1603x over previous
import jax
import jax.numpy as jnp
from jax.experimental import pallas as pl
from jax.experimental.pallas import tpu as pltpu

EPS = 1e-5
K = 16
P = 1024
BIGF = 1e30
_SQRT1_2 = 0.7071067811865476


def _gelu(x):
    return 0.5 * x * (1.0 + jax.lax.erf(x * _SQRT1_2))


def _frame_kernel(x_ref, xT_ref, waT_ref, wbT_ref, s1_ref, c1_ref, w2T_ref,
                  s2_ref, c2_ref, pw1aT_ref, pw1bT_ref, pb1_ref, pg1_ref,
                  pe1_ref, pw2T_ref, pb2_ref, pg2_ref, pe2_ref, o_ref):
    f32 = jnp.float32
    x = x_ref[0]
    xT = xT_ref[0]

    lane8 = jax.lax.broadcasted_iota(jnp.int32, (1, 8), 1)
    sub8 = jax.lax.broadcasted_iota(jnp.int32, (8, 1), 0)
    xyz = jnp.where(lane8 < 3, x, 0.0)
    xyzT = jnp.where(sub8 < 3, xT, 0.0)

    dot = jnp.dot(xyz, xyzT, preferred_element_type=f32)
    sqr = jnp.sum(xyz * xyz, axis=1, keepdims=True)
    sqc = jnp.sum(xyzT * xyzT, axis=0, keepdims=True)
    d2 = jnp.maximum(sqr + sqc - 2.0 * dot, 0.0)

    iota = jax.lax.broadcasted_iota(jnp.int32, (P, P), 1).astype(f32)

    axi = jnp.dot(x, waT_ref[...], preferred_element_type=f32)
    s1 = s1_ref[...]
    c1 = c1_ref[...]
    s2 = s2_ref[...]
    c2 = c2_ref[...]
    wbT = wbT_ref[...]
    w2T = w2T_ref[...]

    def body(_, carry):
        d2c, ymax, ymin = carry
        idx = jnp.argmin(d2c, axis=1).astype(jnp.float32).reshape(P, 1)
        sel1 = iota == idx
        d2c = jnp.where(sel1, BIGF, d2c)
        xj_c = [
            jnp.sum(jnp.where(sel1, xT[c:c + 1, :], 0.0), axis=1, keepdims=True)
            for c in range(4)
        ]
        h1p = (axi + xj_c[0] * wbT[0:1] + xj_c[1] * wbT[1:2]
               + xj_c[2] * wbT[2:3] + xj_c[3] * wbT[3:4])
        h1 = _gelu(h1p * s1 + c1)
        y2 = jnp.dot(h1, w2T, preferred_element_type=f32) * s2 + c2
        return d2c, jnp.maximum(ymax, y2), jnp.minimum(ymin, y2)

    ymax0 = jnp.full((P, 128), -BIGF, f32)
    ymin0 = jnp.full((P, 128), BIGF, f32)
    _, ymax, ymin = jax.lax.fori_loop(0, K, body, (d2, ymax0, ymin0), unroll=2)
    local = jnp.maximum(_gelu(ymax), _gelu(ymin))

    y = (jnp.dot(local, pw1aT_ref[...], preferred_element_type=f32)
         + jnp.dot(x, pw1bT_ref[...], preferred_element_type=f32)
         + pb1_ref[...])
    m = y.mean(axis=1, keepdims=True)
    va = ((y - m) ** 2).mean(axis=1, keepdims=True)
    y = _gelu((y - m) / jnp.sqrt(va + EPS) * pg1_ref[...] + pe1_ref[...])
    z = jnp.dot(y, pw2T_ref[...], preferred_element_type=f32) + pb2_ref[...]
    m = z.mean(axis=1, keepdims=True)
    va = ((z - m) ** 2).mean(axis=1, keepdims=True)
    z = _gelu((z - m) / jnp.sqrt(va + EPS) * pg2_ref[...] + pe2_ref[...])

    o_ref[0, :, 0:256] = jnp.max(z, axis=0, keepdims=True)
    o_ref[0, :, 256:512] = jnp.sum(z, axis=0, keepdims=True) * (1.0 / P)


def _temporal_kernel(pf_ref, projT_ref, projb_ref, wc_ref, cb_ref, bs_ref,
                     bb_ref, hw1T_ref, hb1_ref, hw2T_ref, hb2_ref, o_ref):
    f32 = jnp.float32
    pf = pf_ref[...]
    hs = jnp.dot(pf, projT_ref[...], preferred_element_type=f32) + projb_ref[...]
    rmod = jax.lax.broadcasted_iota(jnp.int32, (64, 1), 0) % 16

    def conv_block(lay, xin):
        w0 = wc_ref[3 * lay]
        w1 = wc_ref[3 * lay + 1]
        w2 = wc_ref[3 * lay + 2]
        xm1 = jnp.where(rmod == 0, 0.0,
                        jnp.concatenate([jnp.zeros((1, 256), f32), xin[:-1]], axis=0))
        xp1 = jnp.where(rmod == 15, 0.0,
                        jnp.concatenate([xin[1:], jnp.zeros((1, 256), f32)], axis=0))
        y = (jnp.dot(xm1, w0, preferred_element_type=f32)
             + jnp.dot(xin, w1, preferred_element_type=f32)
             + jnp.dot(xp1, w2, preferred_element_type=f32)
             + cb_ref[lay:lay + 1])
        return _gelu(y * bs_ref[lay:lay + 1] + bb_ref[lay:lay + 1])

    hs = conv_block(0, hs)
    hs = hs + conv_block(1, hs)
    hs = hs + conv_block(2, hs)
    hs = hs + conv_block(3, hs)

    parts = [jnp.max(hs[16 * b:16 * (b + 1)], axis=0, keepdims=True)
             for b in range(4)]
    mx = jnp.concatenate(parts, axis=0)
    h = _gelu(jnp.dot(mx, hw1T_ref[...], preferred_element_type=f32) + hb1_ref[...])
    o_ref[...] = jnp.dot(h, hw2T_ref[...], preferred_element_type=f32) + hb2_ref[...]


def kernel(x_pt, ec_w1, ec_g1, ec_b1, ec_w2, ec_g2, ec_b2, pm_w1, pm_b1,
           pm_g1, pm_be1, pm_w2, pm_b2, pm_g2, pm_be2, proj_w, proj_b,
           c1_w, c1_b, bn1_g, bn1_b, c2_w, c2_b, bn2_g, bn2_b, c3_w, c3_b,
           bn3_g, bn3_b, c4_w, c4_b, bn4_g, bn4_b, h_w1, h_b1, h_w2, h_b2):
    B, T, Pn, C = x_pt.shape
    BT = B * T
    f32 = jnp.float32
    x = x_pt.reshape(BT, Pn, C)
    x8 = jnp.concatenate([x, jnp.zeros((BT, Pn, 8 - C), f32)], axis=-1)
    xT8 = jnp.transpose(x8, (0, 2, 1))

    rbn = 1.0 / jnp.sqrt(jnp.float32(1.0 + EPS))
    waT = (ec_w1[:, :4] - ec_w1[:, 4:]).T
    waT = jnp.concatenate([waT, jnp.zeros((4, 64), f32)], axis=0)
    wbT = jnp.concatenate([ec_w1[:, 4:].T, jnp.zeros((4, 64), f32)], axis=0)
    s1 = (ec_g1 * rbn)[None]
    c1 = ec_b1[None]
    w2T = ec_w2.T
    s2 = (ec_g2 * rbn)[None]
    c2 = ec_b2[None]
    pw1aT = pm_w1[:, :128].T
    pw1bT = jnp.concatenate([pm_w1[:, 128:].T, jnp.zeros((4, 128), f32)], axis=0)

    rep = lambda s: pl.BlockSpec(s, lambda f: tuple([0] * len(s)))
    per_frame = pl.pallas_call(
        _frame_kernel,
        grid=(BT,),
        in_specs=[
            pl.BlockSpec((1, Pn, 8), lambda f: (f, 0, 0)),
            pl.BlockSpec((1, 8, Pn), lambda f: (f, 0, 0)),
            rep((8, 64)), rep((8, 64)), rep((1, 64)), rep((1, 64)),
            rep((64, 128)), rep((1, 128)), rep((1, 128)),
            rep((128, 128)), rep((8, 128)), rep((1, 128)), rep((1, 128)),
            rep((1, 128)),
            rep((128, 256)), rep((1, 256)), rep((1, 256)), rep((1, 256)),
        ],
        out_specs=pl.BlockSpec((1, 1, 512), lambda f: (f, 0, 0)),
        out_shape=jax.ShapeDtypeStruct((BT, 1, 512), f32),
    )(x8, xT8, waT, wbT, s1, c1, w2T, s2, c2,
      pw1aT, pw1bT, pm_b1[None], pm_g1[None], pm_be1[None],
      pm_w2.T, pm_b2[None], pm_g2[None], pm_be2[None])
    per_frame = per_frame.reshape(BT, 512)

    wc = jnp.stack([w[:, :, dt].T for w in (c1_w, c2_w, c3_w, c4_w)
                    for dt in range(3)], axis=0)
    cb = jnp.stack([c1_b, c2_b, c3_b, c4_b], axis=0)
    bs = jnp.stack([bn1_g, bn2_g, bn3_g, bn4_g], axis=0) * rbn
    bb = jnp.stack([bn1_b, bn2_b, bn3_b, bn4_b], axis=0)

    out = pl.pallas_call(
        _temporal_kernel,
        in_specs=[
            pl.BlockSpec((BT, 512), lambda: (0, 0)),
            pl.BlockSpec((512, 256), lambda: (0, 0)),
            pl.BlockSpec((1, 256), lambda: (0, 0)),
            pl.BlockSpec((12, 256, 256), lambda: (0, 0, 0)),
            pl.BlockSpec((4, 256), lambda: (0, 0)),
            pl.BlockSpec((4, 256), lambda: (0, 0)),
            pl.BlockSpec((4, 256), lambda: (0, 0)),
            pl.BlockSpec((256, 128), lambda: (0, 0)),
            pl.BlockSpec((1, 128), lambda: (0, 0)),
            pl.BlockSpec((128, 32), lambda: (0, 0)),
            pl.BlockSpec((1, 32), lambda: (0, 0)),
        ],
        out_specs=pl.BlockSpec((4, 32), lambda: (0, 0)),
        out_shape=jax.ShapeDtypeStruct((4, 32), f32),
    )(per_frame, proj_w.T, proj_b[None], wc, cb, bs, bb,
      h_w1.T, h_b1[None],
      jnp.concatenate([h_w2.T, jnp.zeros((128, 7), f32)], axis=1),
      jnp.concatenate([h_b2, jnp.zeros((7,), f32)])[None])
    return out[:, :25]


# speedup vs baseline: 3.2790x; 1.0376x over previous
import jax
import jax.numpy as jnp
from jax.experimental import pallas as pl
from jax.experimental.pallas import tpu as pltpu

EPS = 1e-5
K = 16
P = 1024
BIGF = 1e30
_SQRT1_2 = 0.7071067811865476


def _gelu(x):
    return 0.5 * x * (1.0 + jax.lax.erf(x * _SQRT1_2))


def _frame_kernel(x_ref, xT_ref, waT_ref, wbT_ref, s1_ref, c1_ref, w2T_ref,
                  s2_ref, c2_ref, pw1aT_ref, pw1bT_ref, pb1_ref, pg1_ref,
                  pe1_ref, pw2T_ref, pb2_ref, pg2_ref, pe2_ref, o_ref):
    f32 = jnp.float32
    x = x_ref[0]
    xT = xT_ref[0]

    lane8 = jax.lax.broadcasted_iota(jnp.int32, (1, 8), 1)
    sub8 = jax.lax.broadcasted_iota(jnp.int32, (8, 1), 0)
    xyz = jnp.where(lane8 < 3, x, 0.0)
    xyzT = jnp.where(sub8 < 3, xT, 0.0)

    dot = jnp.dot(xyz, xyzT, preferred_element_type=f32)
    sqr = jnp.sum(xyz * xyz, axis=1, keepdims=True)
    sqc = jnp.sum(xyzT * xyzT, axis=0, keepdims=True)
    d2 = jnp.maximum(sqr + sqc - 2.0 * dot, 0.0)

    iota = jax.lax.broadcasted_iota(jnp.int32, (P, P), 1).astype(f32)

    axi = jnp.dot(x, waT_ref[...], preferred_element_type=f32)
    s1 = s1_ref[...]
    c1 = c1_ref[...]
    s2 = s2_ref[...]
    c2 = c2_ref[...]
    wbT = wbT_ref[...]
    w2T = w2T_ref[...]

    def body(_, carry):
        d2c, ymax, ymin = carry
        idx = jnp.argmin(d2c, axis=1).astype(jnp.float32).reshape(P, 1)
        sel1 = iota == idx
        d2c = jnp.where(sel1, BIGF, d2c)
        xj_c = [
            jnp.sum(jnp.where(sel1, xT[c:c + 1, :], 0.0), axis=1, keepdims=True)
            for c in range(4)
        ]
        h1p = (axi + xj_c[0] * wbT[0:1] + xj_c[1] * wbT[1:2]
               + xj_c[2] * wbT[2:3] + xj_c[3] * wbT[3:4])
        h1 = _gelu(h1p * s1 + c1)
        y2 = jnp.dot(h1, w2T, preferred_element_type=f32) * s2 + c2
        return d2c, jnp.maximum(ymax, y2), jnp.minimum(ymin, y2)

    ymax0 = jnp.full((P, 128), -BIGF, f32)
    ymin0 = jnp.full((P, 128), BIGF, f32)
    _, ymax, ymin = jax.lax.fori_loop(0, K, body, (d2, ymax0, ymin0), unroll=4)
    local = jnp.maximum(_gelu(ymax), _gelu(ymin))

    y = (jnp.dot(local, pw1aT_ref[...], preferred_element_type=f32)
         + jnp.dot(x, pw1bT_ref[...], preferred_element_type=f32)
         + pb1_ref[...])
    m = y.mean(axis=1, keepdims=True)
    va = ((y - m) ** 2).mean(axis=1, keepdims=True)
    y = _gelu((y - m) / jnp.sqrt(va + EPS) * pg1_ref[...] + pe1_ref[...])
    z = jnp.dot(y, pw2T_ref[...], preferred_element_type=f32) + pb2_ref[...]
    m = z.mean(axis=1, keepdims=True)
    va = ((z - m) ** 2).mean(axis=1, keepdims=True)
    z = _gelu((z - m) / jnp.sqrt(va + EPS) * pg2_ref[...] + pe2_ref[...])

    o_ref[0, :, 0:256] = jnp.max(z, axis=0, keepdims=True)
    o_ref[0, :, 256:512] = jnp.sum(z, axis=0, keepdims=True) * (1.0 / P)


def _temporal_kernel(pf_ref, projT_ref, projb_ref, wc_ref, cb_ref, bs_ref,
                     bb_ref, hw1T_ref, hb1_ref, hw2T_ref, hb2_ref, o_ref):
    f32 = jnp.float32
    pf = pf_ref[...]
    hs = jnp.dot(pf, projT_ref[...], preferred_element_type=f32) + projb_ref[...]
    rmod = jax.lax.broadcasted_iota(jnp.int32, (64, 1), 0) % 16

    def conv_block(lay, xin):
        w0 = wc_ref[3 * lay]
        w1 = wc_ref[3 * lay + 1]
        w2 = wc_ref[3 * lay + 2]
        xm1 = jnp.where(rmod == 0, 0.0,
                        jnp.concatenate([jnp.zeros((1, 256), f32), xin[:-1]], axis=0))
        xp1 = jnp.where(rmod == 15, 0.0,
                        jnp.concatenate([xin[1:], jnp.zeros((1, 256), f32)], axis=0))
        y = (jnp.dot(xm1, w0, preferred_element_type=f32)
             + jnp.dot(xin, w1, preferred_element_type=f32)
             + jnp.dot(xp1, w2, preferred_element_type=f32)
             + cb_ref[lay:lay + 1])
        return _gelu(y * bs_ref[lay:lay + 1] + bb_ref[lay:lay + 1])

    hs = conv_block(0, hs)
    hs = hs + conv_block(1, hs)
    hs = hs + conv_block(2, hs)
    hs = hs + conv_block(3, hs)

    parts = [jnp.max(hs[16 * b:16 * (b + 1)], axis=0, keepdims=True)
             for b in range(4)]
    mx = jnp.concatenate(parts, axis=0)
    h = _gelu(jnp.dot(mx, hw1T_ref[...], preferred_element_type=f32) + hb1_ref[...])
    o_ref[...] = jnp.dot(h, hw2T_ref[...], preferred_element_type=f32) + hb2_ref[...]


def kernel(x_pt, ec_w1, ec_g1, ec_b1, ec_w2, ec_g2, ec_b2, pm_w1, pm_b1,
           pm_g1, pm_be1, pm_w2, pm_b2, pm_g2, pm_be2, proj_w, proj_b,
           c1_w, c1_b, bn1_g, bn1_b, c2_w, c2_b, bn2_g, bn2_b, c3_w, c3_b,
           bn3_g, bn3_b, c4_w, c4_b, bn4_g, bn4_b, h_w1, h_b1, h_w2, h_b2):
    B, T, Pn, C = x_pt.shape
    BT = B * T
    f32 = jnp.float32
    x = x_pt.reshape(BT, Pn, C)
    x8 = jnp.concatenate([x, jnp.zeros((BT, Pn, 8 - C), f32)], axis=-1)
    xT8 = jnp.transpose(x8, (0, 2, 1))

    rbn = 1.0 / jnp.sqrt(jnp.float32(1.0 + EPS))
    waT = (ec_w1[:, :4] - ec_w1[:, 4:]).T
    waT = jnp.concatenate([waT, jnp.zeros((4, 64), f32)], axis=0)
    wbT = jnp.concatenate([ec_w1[:, 4:].T, jnp.zeros((4, 64), f32)], axis=0)
    s1 = (ec_g1 * rbn)[None]
    c1 = ec_b1[None]
    w2T = ec_w2.T
    s2 = (ec_g2 * rbn)[None]
    c2 = ec_b2[None]
    pw1aT = pm_w1[:, :128].T
    pw1bT = jnp.concatenate([pm_w1[:, 128:].T, jnp.zeros((4, 128), f32)], axis=0)

    rep = lambda s: pl.BlockSpec(s, lambda f: tuple([0] * len(s)))
    per_frame = pl.pallas_call(
        _frame_kernel,
        grid=(BT,),
        in_specs=[
            pl.BlockSpec((1, Pn, 8), lambda f: (f, 0, 0)),
            pl.BlockSpec((1, 8, Pn), lambda f: (f, 0, 0)),
            rep((8, 64)), rep((8, 64)), rep((1, 64)), rep((1, 64)),
            rep((64, 128)), rep((1, 128)), rep((1, 128)),
            rep((128, 128)), rep((8, 128)), rep((1, 128)), rep((1, 128)),
            rep((1, 128)),
            rep((128, 256)), rep((1, 256)), rep((1, 256)), rep((1, 256)),
        ],
        out_specs=pl.BlockSpec((1, 1, 512), lambda f: (f, 0, 0)),
        out_shape=jax.ShapeDtypeStruct((BT, 1, 512), f32),
    )(x8, xT8, waT, wbT, s1, c1, w2T, s2, c2,
      pw1aT, pw1bT, pm_b1[None], pm_g1[None], pm_be1[None],
      pm_w2.T, pm_b2[None], pm_g2[None], pm_be2[None])
    per_frame = per_frame.reshape(BT, 512)

    wc = jnp.stack([w[:, :, dt].T for w in (c1_w, c2_w, c3_w, c4_w)
                    for dt in range(3)], axis=0)
    cb = jnp.stack([c1_b, c2_b, c3_b, c4_b], axis=0)
    bs = jnp.stack([bn1_g, bn2_g, bn3_g, bn4_g], axis=0) * rbn
    bb = jnp.stack([bn1_b, bn2_b, bn3_b, bn4_b], axis=0)

    out = pl.pallas_call(
        _temporal_kernel,
        in_specs=[
            pl.BlockSpec((BT, 512), lambda: (0, 0)),
            pl.BlockSpec((512, 256), lambda: (0, 0)),
            pl.BlockSpec((1, 256), lambda: (0, 0)),
            pl.BlockSpec((12, 256, 256), lambda: (0, 0, 0)),
            pl.BlockSpec((4, 256), lambda: (0, 0)),
            pl.BlockSpec((4, 256), lambda: (0, 0)),
            pl.BlockSpec((4, 256), lambda: (0, 0)),
            pl.BlockSpec((256, 128), lambda: (0, 0)),
            pl.BlockSpec((1, 128), lambda: (0, 0)),
            pl.BlockSpec((128, 32), lambda: (0, 0)),
            pl.BlockSpec((1, 32), lambda: (0, 0)),
        ],
        out_specs=pl.BlockSpec((4, 32), lambda: (0, 0)),
        out_shape=jax.ShapeDtypeStruct((4, 32), f32),
    )(per_frame, proj_w.T, proj_b[None], wc, cb, bs, bb,
      h_w1.T, h_b1[None],
      jnp.concatenate([h_w2.T, jnp.zeros((128, 7), f32)], axis=1),
      jnp.concatenate([h_b2, jnp.zeros((7,), f32)])[None])
    return out[:, :25]


# speedup vs baseline: 3.3342x; 1.0168x over previous
import jax
import jax.numpy as jnp
from jax.experimental import pallas as pl
from jax.experimental.pallas import tpu as pltpu

EPS = 1e-5
K = 16
P = 1024
BIGF = 1e30
_SQRT1_2 = 0.7071067811865476


def _gelu(x):
    return 0.5 * x * (1.0 + jax.lax.erf(x * _SQRT1_2))


def _frame_kernel(x_ref, xT_ref, waT_ref, wbT_ref, s1_ref, c1_ref, w2T_ref,
                  s2_ref, c2_ref, pw1aT_ref, pw1bT_ref, pb1_ref, pg1_ref,
                  pe1_ref, pw2T_ref, pb2_ref, pg2_ref, pe2_ref, o_ref):
    f32 = jnp.float32
    x = x_ref[0]
    xT = xT_ref[0]

    lane8 = jax.lax.broadcasted_iota(jnp.int32, (1, 8), 1)
    sub8 = jax.lax.broadcasted_iota(jnp.int32, (8, 1), 0)
    xyz = jnp.where(lane8 < 3, x, 0.0)
    xyzT = jnp.where(sub8 < 3, xT, 0.0)

    dot = jnp.dot(xyz, xyzT, preferred_element_type=f32)
    sqr = jnp.sum(xyz * xyz, axis=1, keepdims=True)
    sqc = jnp.sum(xyzT * xyzT, axis=0, keepdims=True)
    d2 = jnp.maximum(sqr + sqc - 2.0 * dot, 0.0)

    iota = jax.lax.broadcasted_iota(jnp.int32, (P, P), 1).astype(f32)

    axi = jnp.dot(x, waT_ref[...], preferred_element_type=f32)
    s1 = s1_ref[...]
    c1 = c1_ref[...]
    s2 = s2_ref[...]
    c2 = c2_ref[...]
    wbT = wbT_ref[...]
    w2T = w2T_ref[...]

    def body(_, carry):
        d2c, ymax, ymin = carry
        idx = jnp.argmin(d2c, axis=1).astype(jnp.float32).reshape(P, 1)
        sel1 = iota == idx
        d2c = jnp.where(sel1, BIGF, d2c)
        xj_c = [
            jnp.sum(jnp.where(sel1, xT[c:c + 1, :], 0.0), axis=1, keepdims=True)
            for c in range(4)
        ]
        h1p = (axi + xj_c[0] * wbT[0:1] + xj_c[1] * wbT[1:2]
               + xj_c[2] * wbT[2:3] + xj_c[3] * wbT[3:4])
        h1 = _gelu(h1p * s1 + c1)
        y2 = jnp.dot(h1, w2T, preferred_element_type=f32) * s2 + c2
        return d2c, jnp.maximum(ymax, y2), jnp.minimum(ymin, y2)

    ymax0 = jnp.full((P, 128), -BIGF, f32)
    ymin0 = jnp.full((P, 128), BIGF, f32)
    _, ymax, ymin = jax.lax.fori_loop(0, K, body, (d2, ymax0, ymin0), unroll=8)
    local = jnp.maximum(_gelu(ymax), _gelu(ymin))

    y = (jnp.dot(local, pw1aT_ref[...], preferred_element_type=f32)
         + jnp.dot(x, pw1bT_ref[...], preferred_element_type=f32)
         + pb1_ref[...])
    m = y.mean(axis=1, keepdims=True)
    va = ((y - m) ** 2).mean(axis=1, keepdims=True)
    y = _gelu((y - m) / jnp.sqrt(va + EPS) * pg1_ref[...] + pe1_ref[...])
    z = jnp.dot(y, pw2T_ref[...], preferred_element_type=f32) + pb2_ref[...]
    m = z.mean(axis=1, keepdims=True)
    va = ((z - m) ** 2).mean(axis=1, keepdims=True)
    z = _gelu((z - m) / jnp.sqrt(va + EPS) * pg2_ref[...] + pe2_ref[...])

    o_ref[0, :, 0:256] = jnp.max(z, axis=0, keepdims=True)
    o_ref[0, :, 256:512] = jnp.sum(z, axis=0, keepdims=True) * (1.0 / P)


def _temporal_kernel(pf_ref, projT_ref, projb_ref, wc_ref, cb_ref, bs_ref,
                     bb_ref, hw1T_ref, hb1_ref, hw2T_ref, hb2_ref, o_ref):
    f32 = jnp.float32
    pf = pf_ref[...]
    hs = jnp.dot(pf, projT_ref[...], preferred_element_type=f32) + projb_ref[...]
    rmod = jax.lax.broadcasted_iota(jnp.int32, (64, 1), 0) % 16

    def conv_block(lay, xin):
        w0 = wc_ref[3 * lay]
        w1 = wc_ref[3 * lay + 1]
        w2 = wc_ref[3 * lay + 2]
        xm1 = jnp.where(rmod == 0, 0.0,
                        jnp.concatenate([jnp.zeros((1, 256), f32), xin[:-1]], axis=0))
        xp1 = jnp.where(rmod == 15, 0.0,
                        jnp.concatenate([xin[1:], jnp.zeros((1, 256), f32)], axis=0))
        y = (jnp.dot(xm1, w0, preferred_element_type=f32)
             + jnp.dot(xin, w1, preferred_element_type=f32)
             + jnp.dot(xp1, w2, preferred_element_type=f32)
             + cb_ref[lay:lay + 1])
        return _gelu(y * bs_ref[lay:lay + 1] + bb_ref[lay:lay + 1])

    hs = conv_block(0, hs)
    hs = hs + conv_block(1, hs)
    hs = hs + conv_block(2, hs)
    hs = hs + conv_block(3, hs)

    parts = [jnp.max(hs[16 * b:16 * (b + 1)], axis=0, keepdims=True)
             for b in range(4)]
    mx = jnp.concatenate(parts, axis=0)
    h = _gelu(jnp.dot(mx, hw1T_ref[...], preferred_element_type=f32) + hb1_ref[...])
    o_ref[...] = jnp.dot(h, hw2T_ref[...], preferred_element_type=f32) + hb2_ref[...]


def kernel(x_pt, ec_w1, ec_g1, ec_b1, ec_w2, ec_g2, ec_b2, pm_w1, pm_b1,
           pm_g1, pm_be1, pm_w2, pm_b2, pm_g2, pm_be2, proj_w, proj_b,
           c1_w, c1_b, bn1_g, bn1_b, c2_w, c2_b, bn2_g, bn2_b, c3_w, c3_b,
           bn3_g, bn3_b, c4_w, c4_b, bn4_g, bn4_b, h_w1, h_b1, h_w2, h_b2):
    B, T, Pn, C = x_pt.shape
    BT = B * T
    f32 = jnp.float32
    x = x_pt.reshape(BT, Pn, C)
    x8 = jnp.concatenate([x, jnp.zeros((BT, Pn, 8 - C), f32)], axis=-1)
    xT8 = jnp.transpose(x8, (0, 2, 1))

    rbn = 1.0 / jnp.sqrt(jnp.float32(1.0 + EPS))
    waT = (ec_w1[:, :4] - ec_w1[:, 4:]).T
    waT = jnp.concatenate([waT, jnp.zeros((4, 64), f32)], axis=0)
    wbT = jnp.concatenate([ec_w1[:, 4:].T, jnp.zeros((4, 64), f32)], axis=0)
    s1 = (ec_g1 * rbn)[None]
    c1 = ec_b1[None]
    w2T = ec_w2.T
    s2 = (ec_g2 * rbn)[None]
    c2 = ec_b2[None]
    pw1aT = pm_w1[:, :128].T
    pw1bT = jnp.concatenate([pm_w1[:, 128:].T, jnp.zeros((4, 128), f32)], axis=0)

    rep = lambda s: pl.BlockSpec(s, lambda f: tuple([0] * len(s)))
    per_frame = pl.pallas_call(
        _frame_kernel,
        grid=(BT,),
        in_specs=[
            pl.BlockSpec((1, Pn, 8), lambda f: (f, 0, 0)),
            pl.BlockSpec((1, 8, Pn), lambda f: (f, 0, 0)),
            rep((8, 64)), rep((8, 64)), rep((1, 64)), rep((1, 64)),
            rep((64, 128)), rep((1, 128)), rep((1, 128)),
            rep((128, 128)), rep((8, 128)), rep((1, 128)), rep((1, 128)),
            rep((1, 128)),
            rep((128, 256)), rep((1, 256)), rep((1, 256)), rep((1, 256)),
        ],
        out_specs=pl.BlockSpec((1, 1, 512), lambda f: (f, 0, 0)),
        out_shape=jax.ShapeDtypeStruct((BT, 1, 512), f32),
    )(x8, xT8, waT, wbT, s1, c1, w2T, s2, c2,
      pw1aT, pw1bT, pm_b1[None], pm_g1[None], pm_be1[None],
      pm_w2.T, pm_b2[None], pm_g2[None], pm_be2[None])
    per_frame = per_frame.reshape(BT, 512)

    wc = jnp.stack([w[:, :, dt].T for w in (c1_w, c2_w, c3_w, c4_w)
                    for dt in range(3)], axis=0)
    cb = jnp.stack([c1_b, c2_b, c3_b, c4_b], axis=0)
    bs = jnp.stack([bn1_g, bn2_g, bn3_g, bn4_g], axis=0) * rbn
    bb = jnp.stack([bn1_b, bn2_b, bn3_b, bn4_b], axis=0)

    out = pl.pallas_call(
        _temporal_kernel,
        in_specs=[
            pl.BlockSpec((BT, 512), lambda: (0, 0)),
            pl.BlockSpec((512, 256), lambda: (0, 0)),
            pl.BlockSpec((1, 256), lambda: (0, 0)),
            pl.BlockSpec((12, 256, 256), lambda: (0, 0, 0)),
            pl.BlockSpec((4, 256), lambda: (0, 0)),
            pl.BlockSpec((4, 256), lambda: (0, 0)),
            pl.BlockSpec((4, 256), lambda: (0, 0)),
            pl.BlockSpec((256, 128), lambda: (0, 0)),
            pl.BlockSpec((1, 128), lambda: (0, 0)),
            pl.BlockSpec((128, 32), lambda: (0, 0)),
            pl.BlockSpec((1, 32), lambda: (0, 0)),
        ],
        out_specs=pl.BlockSpec((4, 32), lambda: (0, 0)),
        out_shape=jax.ShapeDtypeStruct((4, 32), f32),
    )(per_frame, proj_w.T, proj_b[None], wc, cb, bs, bb,
      h_w1.T, h_b1[None],
      jnp.concatenate([h_w2.T, jnp.zeros((128, 7), f32)], axis=1),
      jnp.concatenate([h_b2, jnp.zeros((7,), f32)])[None])
    return out[:, :25]
